# Initial kernel scaffold; baseline (speedup 1.0000x reference)
import jax
import jax.numpy as jnp
from jax.experimental import pallas as pl


def kernel(x, edge_index, W_in, b_in, lin_l_W, lin_l_b, lin_r_W, lin_r_b, att_W, gat_b, W_ih, W_hh, b_ih, b_hh, W_out, b_out):
    raise NotImplementedError("write your pallas kernel here")


# speedup vs baseline: 1.6162x; 1.6162x over previous
import functools

import jax
import jax.numpy as jnp
from jax import lax
from jax.experimental import pallas as pl
from jax.experimental.pallas import tpu as pltpu

N_NODES = 10000
HID = 128
OUT_CH = 64
N_LAYERS = 3

ROW_BLK = 1000


def _prologue_body(x_ref, Win_ref, bin_ref, Wl_ref, bl_ref, Wr_ref, br_ref,
                   xl_ref, xr_ref):
    h = jnp.maximum(
        jnp.dot(x_ref[...], Win_ref[...].T, preferred_element_type=jnp.float32)
        + bin_ref[...], 0.0)
    xl_ref[...] = jnp.dot(h, Wl_ref[...].T, preferred_element_type=jnp.float32) + bl_ref[...]
    xr_ref[...] = jnp.dot(h, Wr_ref[...].T, preferred_element_type=jnp.float32) + br_ref[...]


def _prologue(x, W_in, b_in, Wl, bl, Wr, br):
    grid = (N_NODES // ROW_BLK,)
    blk = lambda i: (i, 0)
    full = lambda i: (0, 0)
    return pl.pallas_call(
        _prologue_body,
        grid=grid,
        in_specs=[
            pl.BlockSpec((ROW_BLK, HID), blk),
            pl.BlockSpec((HID, HID), full),
            pl.BlockSpec((1, HID), full),
            pl.BlockSpec((HID, HID), full),
            pl.BlockSpec((1, HID), full),
            pl.BlockSpec((HID, HID), full),
            pl.BlockSpec((1, HID), full),
        ],
        out_specs=[
            pl.BlockSpec((ROW_BLK, HID), blk),
            pl.BlockSpec((ROW_BLK, HID), blk),
        ],
        out_shape=[
            jax.ShapeDtypeStruct((N_NODES, HID), jnp.float32),
            jax.ShapeDtypeStruct((N_NODES, HID), jnp.float32),
        ],
    )(x, W_in, b_in.reshape(1, HID), Wl, bl.reshape(1, HID), Wr, br.reshape(1, HID))


def _mid_body(o0_ref, o1_ref, gb_ref, Wl_ref, bl_ref, Wr_ref, br_ref,
              xl_ref, xr_ref):
    v = o0_ref[...] + o1_ref[...] + gb_ref[...]
    h = jnp.where(v > 0.0, v, jnp.exp(jnp.minimum(v, 0.0)) - 1.0)
    xl_ref[...] = jnp.dot(h, Wl_ref[...].T, preferred_element_type=jnp.float32) + bl_ref[...]
    xr_ref[...] = jnp.dot(h, Wr_ref[...].T, preferred_element_type=jnp.float32) + br_ref[...]


def _mid(out0, out1, gb, Wl, bl, Wr, br):
    grid = (N_NODES // ROW_BLK,)
    blk = lambda i: (i, 0)
    full = lambda i: (0, 0)
    return pl.pallas_call(
        _mid_body,
        grid=grid,
        in_specs=[
            pl.BlockSpec((ROW_BLK, HID), blk),
            pl.BlockSpec((ROW_BLK, HID), blk),
            pl.BlockSpec((1, HID), full),
            pl.BlockSpec((HID, HID), full),
            pl.BlockSpec((1, HID), full),
            pl.BlockSpec((HID, HID), full),
            pl.BlockSpec((1, HID), full),
        ],
        out_specs=[
            pl.BlockSpec((ROW_BLK, HID), blk),
            pl.BlockSpec((ROW_BLK, HID), blk),
        ],
        out_shape=[
            jax.ShapeDtypeStruct((N_NODES, HID), jnp.float32),
            jax.ShapeDtypeStruct((N_NODES, HID), jnp.float32),
        ],
    )(out0, out1, gb.reshape(1, HID), Wl, bl.reshape(1, HID), Wr, br.reshape(1, HID))


def _epilogue_body(o0_ref, o1_ref, gb_ref, Wih_ref, bih_ref, Wout_ref, bout_ref,
                   emb_ref, out_ref, acc_ref):
    i = pl.program_id(0)
    h = o0_ref[...] + o1_ref[...] + gb_ref[...]
    emb_ref[...] = h
    gates = jnp.dot(h, Wih_ref[...].T, preferred_element_type=jnp.float32) + bih_ref[...]
    i_g = gates[:, 0 * HID:1 * HID]
    f_g = gates[:, 1 * HID:2 * HID]
    g_g = gates[:, 2 * HID:3 * HID]
    o_g = gates[:, 3 * HID:4 * HID]
    del f_g
    c = jax.nn.sigmoid(i_g) * jnp.tanh(g_g)
    ht = jax.nn.sigmoid(o_g) * jnp.tanh(c)

    @pl.when(i == 0)
    def _():
        acc_ref[...] = jnp.zeros_like(acc_ref)

    acc_ref[...] += jnp.sum(ht, axis=0, keepdims=True)

    @pl.when(i == pl.num_programs(0) - 1)
    def _():
        xg = acc_ref[...] * (1.0 / N_NODES)
        out_ref[...] = jnp.dot(xg, Wout_ref[...].T,
                               preferred_element_type=jnp.float32) + bout_ref[...]


def _epilogue(out0, out1, gb, W_ih, b_ih, b_hh, W_out, b_out):
    grid = (N_NODES // ROW_BLK,)
    blk = lambda i: (i, 0)
    full = lambda i: (0, 0)
    return pl.pallas_call(
        _epilogue_body,
        grid=grid,
        in_specs=[
            pl.BlockSpec((ROW_BLK, HID), blk),
            pl.BlockSpec((ROW_BLK, HID), blk),
            pl.BlockSpec((1, HID), full),
            pl.BlockSpec((4 * HID, HID), full),
            pl.BlockSpec((1, 4 * HID), full),
            pl.BlockSpec((OUT_CH, HID), full),
            pl.BlockSpec((1, OUT_CH), full),
        ],
        out_specs=[
            pl.BlockSpec((ROW_BLK, HID), blk),
            pl.BlockSpec((1, OUT_CH), full),
        ],
        out_shape=[
            jax.ShapeDtypeStruct((N_NODES, HID), jnp.float32),
            jax.ShapeDtypeStruct((1, OUT_CH), jnp.float32),
        ],
        scratch_shapes=[pltpu.VMEM((1, HID), jnp.float32)],
    )(out0, out1, gb.reshape(1, HID), W_ih, (b_ih + b_hh).reshape(1, 4 * HID),
      W_out, b_out.reshape(1, OUT_CH))


def _edge_phase(xl, xr, src, dst, att):
    u = xr[dst] + xl[src]
    e = jnp.where(u > 0, u, 0.2 * u)
    alpha = e @ att
    ex = jnp.exp(alpha)
    denom = jax.ops.segment_sum(ex, dst, num_segments=N_NODES)
    alpha_n = ex / (denom[dst] + 1e-16)
    out = jax.ops.segment_sum(xl[src] * alpha_n[:, None], dst,
                              num_segments=N_NODES)
    return out, alpha_n


def kernel(x, edge_index, W_in, b_in, lin_l_W, lin_l_b, lin_r_W, lin_r_b,
           att_W, gat_b, W_ih, W_hh, b_ih, b_hh, W_out, b_out):
    del W_hh
    N = x.shape[0]
    loop = jnp.arange(N, dtype=edge_index.dtype)
    src = jnp.concatenate([edge_index[0], loop]).astype(jnp.int32)
    dst = jnp.concatenate([edge_index[1], loop]).astype(jnp.int32)

    attn = []
    xl, xr = _prologue(x, W_in, b_in, lin_l_W[0], lin_l_b[0], lin_r_W[0],
                       lin_r_b[0])
    for i in range(N_LAYERS):
        out, alpha = _edge_phase(xl, xr, src, dst, att_W[i, 0])
        attn.append(alpha[:, None])
        if i < N_LAYERS - 1:
            xl, xr = _mid(out, jnp.zeros_like(out), gat_b[i], lin_l_W[i + 1],
                          lin_l_b[i + 1], lin_r_W[i + 1], lin_r_b[i + 1])
        else:
            emb, final = _epilogue(out, jnp.zeros_like(out), gat_b[i], W_ih,
                                   b_ih, b_hh, W_out, b_out)
    return (final, emb) + tuple(attn)


# speedup vs baseline: 8.8814x; 5.4953x over previous
import jax
import jax.numpy as jnp
from jax import lax
from jax.experimental import pallas as pl
from jax.experimental.pallas import tpu as pltpu
from jax.experimental.pallas import tpu_sc as plsc

N_NODES = 10000
N_P = 10240
HID = 128
OUT_CH = 64
N_LAYERS = 3

E_VALID = 330000
NC = 2
NS = 16
NW = NC * NS
EB = 128
NB = 81
EPW = EB * NB
E_PAD = EPW * NW
NPT = N_P // NS

ROW_BLK = 1024


def _edge_body(xl_hbm, xr_hbm, src_hbm, dst_hbm, att_hbm,
               ex_hbm, den_hbm, outu_hbm,
               src_v, dst_v, lrows, rrows, att_v, ex_v,
               den_sh, outu_sh, sem1, sem2):
    cid = lax.axis_index("c")
    sid = lax.axis_index("s")
    base = (cid * NS + sid) * EPW
    iota = lax.iota(jnp.int32, 16)
    zero16 = jnp.zeros((16,), jnp.float32)
    perms = [jnp.bitwise_xor(iota, sh) for sh in (8, 4, 2, 1)]

    def zex(i, carry):
        ex_v[pl.ds(i * 16, 16)] = zero16
        return carry
    lax.fori_loop(0, EB // 16, zex, 0)
    for j in range(NPT // EB):
        pltpu.sync_copy(ex_v, den_sh.at[pl.ds(sid * NPT + j * EB, EB)])

    def zrow(i, carry):
        for k in range(8):
            lrows[i, pl.ds(k * 16, 16)] = zero16
        return carry
    lax.fori_loop(0, EB, zrow, 0)
    for j in range(NPT // EB):
        pltpu.sync_copy(lrows, outu_sh.at[pl.ds(sid * NPT + j * EB, EB)])
    pltpu.sync_copy(att_hbm, att_v)
    plsc.subcore_barrier()
    av = [att_v[pl.ds(cc * 16, 16)] for cc in range(8)]

    def batch(i, carry):
        eb = base + i * EB
        pltpu.sync_copy(src_hbm.at[pl.ds(eb, EB)], src_v)
        pltpu.sync_copy(dst_hbm.at[pl.ds(eb, EB)], dst_v)
        pltpu.async_copy(xl_hbm.at[src_v], lrows, sem1).wait()
        pltpu.async_copy(xr_hbm.at[dst_v], rrows, sem2).wait()

        def group(g, gcarry):
            exg = zero16
            for e in range(16):
                row = g * 16 + e
                acc = zero16
                for cc in range(8):
                    l = lrows[row, pl.ds(cc * 16, 16)]
                    r = rrows[row, pl.ds(cc * 16, 16)]
                    u = l + r
                    acc = acc + av[cc] * jnp.maximum(u, 0.2 * u)
                for perm in perms:
                    acc = acc + acc[perm]
                exg = jnp.where(iota == e, acc, exg)
            eid = eb + g * 16 + iota
            ex = jnp.where(eid < E_VALID, jnp.exp(exg), 0.0)
            ex_v[pl.ds(g * 16, 16)] = ex
            for e in range(16):
                row = g * 16 + e
                a = ex[e]
                for k in range(8):
                    lrows[row, pl.ds(k * 16, 16)] = \
                        lrows[row, pl.ds(k * 16, 16)] * a
            return gcarry

        lax.fori_loop(0, EB // 16, group, 0)
        pltpu.sync_copy(ex_v, ex_hbm.at[pl.ds(eb, EB)])
        pltpu.sync_copy(ex_v, den_sh.at[dst_v], add=True)
        pltpu.sync_copy(lrows, outu_sh.at[dst_v], add=True)
        return carry

    lax.fori_loop(0, NB, batch, 0)

    plsc.subcore_barrier()
    pltpu.sync_copy(den_sh.at[pl.ds(sid * NPT, NPT)],
                    den_hbm.at[pl.ds(cid * N_P + sid * NPT, NPT)])
    pltpu.sync_copy(outu_sh.at[pl.ds(sid * NPT, NPT)],
                    outu_hbm.at[pl.ds(cid * N_P + sid * NPT, NPT)])


def _edge_phase(xl, xr, src, dst, att):
    f = pl.kernel(
        _edge_body,
        out_type=[
            jax.ShapeDtypeStruct((E_PAD,), jnp.float32),
            jax.ShapeDtypeStruct((NC * N_P,), jnp.float32),
            jax.ShapeDtypeStruct((NC * N_P, HID), jnp.float32),
        ],
        mesh=plsc.VectorSubcoreMesh(core_axis_name="c", subcore_axis_name="s"),
        scratch_types=[
            pltpu.VMEM((EB,), jnp.int32),
            pltpu.VMEM((EB,), jnp.int32),
            pltpu.VMEM((EB, HID), jnp.float32),
            pltpu.VMEM((EB, HID), jnp.float32),
            pltpu.VMEM((HID,), jnp.float32),
            pltpu.VMEM((EB,), jnp.float32),
            pltpu.VMEM_SHARED((N_P,), jnp.float32),
            pltpu.VMEM_SHARED((N_P, HID), jnp.float32),
            pltpu.SemaphoreType.DMA,
            pltpu.SemaphoreType.DMA,
        ],
    )
    return f(xl, xr, src, dst, att)


def _alpha_body(ex0, ex1, ex2, dst_hbm, d0, d1, d2,
                al0, al1, al2,
                dst_v, ex_v, dv, al_v, sem):
    cid = lax.axis_index("c")
    sid = lax.axis_index("s")
    base = (cid * NS + sid) * EPW

    for ex_hbm, dsum_hbm, al_hbm in ((ex0, d0, al0), (ex1, d1, al1),
                                     (ex2, d2, al2)):
        def batch(i, carry):
            eb = base + i * EB
            pltpu.sync_copy(dst_hbm.at[pl.ds(eb, EB)], dst_v)
            pltpu.sync_copy(ex_hbm.at[pl.ds(eb, EB)], ex_v)
            pltpu.async_copy(dsum_hbm.at[dst_v], dv, sem).wait()

            def group(g, gcarry):
                ex16 = ex_v[pl.ds(g * 16, 16)]
                d16 = dv[pl.ds(g * 16, 16)]
                al_v[pl.ds(g * 16, 16)] = ex16 / (d16 + 1e-16)
                return gcarry

            lax.fori_loop(0, EB // 16, group, 0)
            pltpu.sync_copy(al_v, al_hbm.at[pl.ds(eb, EB)])
            return carry

        lax.fori_loop(0, NB, batch, 0)


def _alpha3(exs, dst, dsums):
    f = pl.kernel(
        _alpha_body,
        out_type=[jax.ShapeDtypeStruct((E_PAD,), jnp.float32)] * 3,
        mesh=plsc.VectorSubcoreMesh(core_axis_name="c", subcore_axis_name="s"),
        scratch_types=[
            pltpu.VMEM((EB,), jnp.int32),
            pltpu.VMEM((EB,), jnp.float32),
            pltpu.VMEM((EB,), jnp.float32),
            pltpu.VMEM((EB,), jnp.float32),
            pltpu.SemaphoreType.DMA,
        ],
    )
    return f(exs[0], exs[1], exs[2], dst, dsums[0], dsums[1], dsums[2])


def _prologue_body(x_ref, Win_ref, bin_ref, Wl_ref, bl_ref, Wr_ref, br_ref,
                   xl_ref, xr_ref):
    h = jnp.maximum(
        jnp.dot(x_ref[...], Win_ref[...].T, preferred_element_type=jnp.float32)
        + bin_ref[...], 0.0)
    xl_ref[...] = jnp.dot(h, Wl_ref[...].T, preferred_element_type=jnp.float32) + bl_ref[...]
    xr_ref[...] = jnp.dot(h, Wr_ref[...].T, preferred_element_type=jnp.float32) + br_ref[...]


def _prologue(x, W_in, b_in, Wl, bl, Wr, br):
    grid = (N_P // ROW_BLK,)
    blk = lambda i: (i, 0)
    full = lambda i: (0, 0)
    return pl.pallas_call(
        _prologue_body,
        grid=grid,
        in_specs=[
            pl.BlockSpec((ROW_BLK, HID), blk),
            pl.BlockSpec((HID, HID), full),
            pl.BlockSpec((1, HID), full),
            pl.BlockSpec((HID, HID), full),
            pl.BlockSpec((1, HID), full),
            pl.BlockSpec((HID, HID), full),
            pl.BlockSpec((1, HID), full),
        ],
        out_specs=[
            pl.BlockSpec((ROW_BLK, HID), blk),
            pl.BlockSpec((ROW_BLK, HID), blk),
        ],
        out_shape=[
            jax.ShapeDtypeStruct((N_P, HID), jnp.float32),
            jax.ShapeDtypeStruct((N_P, HID), jnp.float32),
        ],
    )(x, W_in, b_in.reshape(1, HID), Wl, bl.reshape(1, HID), Wr, br.reshape(1, HID))


def _combine(o0_ref, o1_ref, den_ref):
    s = jnp.sum(den_ref[...], axis=(0, 1))
    inv = 1.0 / (s + 1e-16)
    return (o0_ref[...] + o1_ref[...]) * inv[:, None], s


def _mid_body(o0_ref, o1_ref, den_ref, gb_ref, Wl_ref, bl_ref, Wr_ref, br_ref,
              xl_ref, xr_ref, dsum_ref):
    out, s = _combine(o0_ref, o1_ref, den_ref)
    dsum_ref[...] = s[None, None, :]
    v = out + gb_ref[...]
    h = jnp.where(v > 0.0, v, jnp.exp(jnp.minimum(v, 0.0)) - 1.0)
    xl_ref[...] = jnp.dot(h, Wl_ref[...].T, preferred_element_type=jnp.float32) + bl_ref[...]
    xr_ref[...] = jnp.dot(h, Wr_ref[...].T, preferred_element_type=jnp.float32) + br_ref[...]


def _mid(outu, den, gb, Wl, bl, Wr, br):
    grid = (N_P // ROW_BLK,)
    blk = lambda i: (i, 0)
    blk1 = lambda i: (i + N_P // ROW_BLK, 0)
    dblk = lambda i: (0, 0, i)
    full = lambda i: (0, 0)
    return pl.pallas_call(
        _mid_body,
        grid=grid,
        in_specs=[
            pl.BlockSpec((ROW_BLK, HID), blk),
            pl.BlockSpec((ROW_BLK, HID), blk1),
            pl.BlockSpec((NC, 1, ROW_BLK), dblk),
            pl.BlockSpec((1, HID), full),
            pl.BlockSpec((HID, HID), full),
            pl.BlockSpec((1, HID), full),
            pl.BlockSpec((HID, HID), full),
            pl.BlockSpec((1, HID), full),
        ],
        out_specs=[
            pl.BlockSpec((ROW_BLK, HID), blk),
            pl.BlockSpec((ROW_BLK, HID), blk),
            pl.BlockSpec((1, 1, ROW_BLK), dblk),
        ],
        out_shape=[
            jax.ShapeDtypeStruct((N_P, HID), jnp.float32),
            jax.ShapeDtypeStruct((N_P, HID), jnp.float32),
            jax.ShapeDtypeStruct((1, 1, N_P), jnp.float32),
        ],
    )(outu, outu, den, gb.reshape(1, HID), Wl, bl.reshape(1, HID),
      Wr, br.reshape(1, HID))


def _epilogue_body(o0_ref, o1_ref, den_ref, gb_ref, Wih_ref, bih_ref,
                   Wout_ref, bout_ref,
                   emb_ref, out_ref, dsum_ref, acc_ref):
    i = pl.program_id(0)
    out, s = _combine(o0_ref, o1_ref, den_ref)
    dsum_ref[...] = s[None, None, :]
    h = out + gb_ref[...]
    emb_ref[...] = h
    gates = jnp.dot(h, Wih_ref[...].T, preferred_element_type=jnp.float32) + bih_ref[...]
    i_g = gates[:, 0 * HID:1 * HID]
    g_g = gates[:, 2 * HID:3 * HID]
    o_g = gates[:, 3 * HID:4 * HID]
    c = jax.nn.sigmoid(i_g) * jnp.tanh(g_g)
    ht = jax.nn.sigmoid(o_g) * jnp.tanh(c)
    rows = lax.broadcasted_iota(jnp.int32, (ROW_BLK, 1), 0) + i * ROW_BLK
    ht = jnp.where(rows < N_NODES, ht, 0.0)

    @pl.when(i == 0)
    def _():
        acc_ref[...] = jnp.zeros_like(acc_ref)

    acc_ref[...] += jnp.sum(ht, axis=0, keepdims=True)

    @pl.when(i == pl.num_programs(0) - 1)
    def _():
        xg = acc_ref[...] * (1.0 / N_NODES)
        out_ref[...] = jnp.dot(xg, Wout_ref[...].T,
                               preferred_element_type=jnp.float32) + bout_ref[...]


def _epilogue(outu, den, gb, W_ih, b_ih, b_hh, W_out, b_out):
    grid = (N_P // ROW_BLK,)
    blk = lambda i: (i, 0)
    blk1 = lambda i: (i + N_P // ROW_BLK, 0)
    dblk = lambda i: (0, 0, i)
    full = lambda i: (0, 0)
    return pl.pallas_call(
        _epilogue_body,
        grid=grid,
        in_specs=[
            pl.BlockSpec((ROW_BLK, HID), blk),
            pl.BlockSpec((ROW_BLK, HID), blk1),
            pl.BlockSpec((NC, 1, ROW_BLK), dblk),
            pl.BlockSpec((1, HID), full),
            pl.BlockSpec((4 * HID, HID), full),
            pl.BlockSpec((1, 4 * HID), full),
            pl.BlockSpec((OUT_CH, HID), full),
            pl.BlockSpec((1, OUT_CH), full),
        ],
        out_specs=[
            pl.BlockSpec((ROW_BLK, HID), blk),
            pl.BlockSpec((1, OUT_CH), full),
            pl.BlockSpec((1, 1, ROW_BLK), dblk),
        ],
        out_shape=[
            jax.ShapeDtypeStruct((N_P, HID), jnp.float32),
            jax.ShapeDtypeStruct((1, OUT_CH), jnp.float32),
            jax.ShapeDtypeStruct((1, 1, N_P), jnp.float32),
        ],
        scratch_shapes=[pltpu.VMEM((1, HID), jnp.float32)],
    )(outu, outu, den, gb.reshape(1, HID), W_ih,
      (b_ih + b_hh).reshape(1, 4 * HID), W_out, b_out.reshape(1, OUT_CH))


def kernel(x, edge_index, W_in, b_in, lin_l_W, lin_l_b, lin_r_W, lin_r_b,
           att_W, gat_b, W_ih, W_hh, b_ih, b_hh, W_out, b_out):
    del W_hh
    N = x.shape[0]
    loop = jnp.arange(N, dtype=jnp.int32)
    pad = jnp.zeros((E_PAD - E_VALID,), jnp.int32)
    src = jnp.concatenate([edge_index[0].astype(jnp.int32), loop, pad])
    dst = jnp.concatenate([edge_index[1].astype(jnp.int32), loop, pad])
    x_p = jnp.concatenate(
        [x, jnp.zeros((N_P - N_NODES, HID), jnp.float32)], axis=0)

    xl, xr = _prologue(x_p, W_in, b_in, lin_l_W[0], lin_l_b[0], lin_r_W[0],
                       lin_r_b[0])
    exs, dsums = [], []
    for i in range(N_LAYERS):
        ex, den, outu = _edge_phase(xl, xr, src, dst, att_W[i, 0])
        den = den.reshape(NC, 1, N_P)
        exs.append(ex)
        if i < N_LAYERS - 1:
            xl, xr, dsum = _mid(outu, den, gat_b[i], lin_l_W[i + 1],
                                lin_l_b[i + 1], lin_r_W[i + 1],
                                lin_r_b[i + 1])
        else:
            emb, final, dsum = _epilogue(outu, den, gat_b[i], W_ih, b_ih,
                                         b_hh, W_out, b_out)
        dsums.append(dsum.reshape(N_P))
    als = _alpha3(exs, dst, dsums)
    attn = tuple(a[:E_VALID, None] for a in als)
    return (final, emb[:N_NODES]) + attn


# speedup vs baseline: 10.2959x; 1.1593x over previous
import jax
import jax.numpy as jnp
from jax import lax
from jax.experimental import pallas as pl
from jax.experimental.pallas import tpu as pltpu
from jax.experimental.pallas import tpu_sc as plsc

N_NODES = 10000
N_P = 10240
HID = 128
OUT_CH = 64
N_LAYERS = 3

E_VALID = 330000
NC = 2
NS = 16
NW = NC * NS
EB = 128
NB = 81
EPW = EB * NB
E_PAD = EPW * NW
NPT = N_P // NS

ROW_BLK = 1024


def _edge_body(xl_hbm, xr_hbm, src_hbm, dst_hbm, att_hbm,
               ex_hbm, den_hbm, outu_hbm,
               src_v, dst_v, lrows, rrows, att_v, ex_v,
               den_sh, outu_sh, sem1, sem2):
    cid = lax.axis_index("c")
    sid = lax.axis_index("s")
    base = (cid * NS + sid) * EPW
    iota = lax.iota(jnp.int32, 16)
    zero16 = jnp.zeros((16,), jnp.float32)
    perms = [jnp.bitwise_xor(iota, sh) for sh in (8, 4, 2, 1)]
    masks = [iota == e for e in range(16)]

    def zex(i, carry):
        ex_v[pl.ds(i * 16, 16)] = zero16
        return carry
    lax.fori_loop(0, EB // 16, zex, 0)
    for j in range(NPT // EB):
        pltpu.sync_copy(ex_v, den_sh.at[pl.ds(sid * NPT + j * EB, EB)])

    def zrow(i, carry):
        for k in range(8):
            lrows[i, pl.ds(k * 16, 16)] = zero16
        return carry
    lax.fori_loop(0, EB, zrow, 0)
    for j in range(NPT // EB):
        pltpu.sync_copy(lrows, outu_sh.at[pl.ds(sid * NPT + j * EB, EB)])
    pltpu.sync_copy(att_hbm, att_v)
    plsc.subcore_barrier()
    av = [att_v[pl.ds(cc * 16, 16)] for cc in range(8)]

    def batch(i, carry):
        eb = base + i * EB
        pltpu.sync_copy(src_hbm.at[pl.ds(eb, EB)], src_v)
        pltpu.sync_copy(dst_hbm.at[pl.ds(eb, EB)], dst_v)
        pltpu.async_copy(xl_hbm.at[src_v], lrows, sem1).wait()
        pltpu.async_copy(xr_hbm.at[dst_v], rrows, sem2).wait()

        def group(g, gcarry):
            exg = zero16
            gbase = eb + g * 16
            for e in range(16):
                row = g * 16 + e
                acc = zero16
                lv = []
                for cc in range(8):
                    l = lrows[row, pl.ds(cc * 16, 16)]
                    lv.append(l)
                    u = l + rrows[row, pl.ds(cc * 16, 16)]
                    acc = acc + av[cc] * jnp.maximum(u, 0.2 * u)
                for perm in perms:
                    acc = acc + acc[perm]
                exb = jnp.where(gbase + e < E_VALID, jnp.exp(acc), 0.0)
                exg = jnp.where(masks[e], exb, exg)
                for cc in range(8):
                    lrows[row, pl.ds(cc * 16, 16)] = lv[cc] * exb
            ex_v[pl.ds(g * 16, 16)] = exg
            return gcarry

        lax.fori_loop(0, EB // 16, group, 0)
        pltpu.sync_copy(ex_v, ex_hbm.at[pl.ds(eb, EB)])
        pltpu.sync_copy(ex_v, den_sh.at[dst_v], add=True)
        pltpu.sync_copy(lrows, outu_sh.at[dst_v], add=True)
        return carry

    lax.fori_loop(0, NB, batch, 0)

    plsc.subcore_barrier()
    pltpu.sync_copy(den_sh.at[pl.ds(sid * NPT, NPT)],
                    den_hbm.at[pl.ds(cid * N_P + sid * NPT, NPT)])
    pltpu.sync_copy(outu_sh.at[pl.ds(sid * NPT, NPT)],
                    outu_hbm.at[pl.ds(cid * N_P + sid * NPT, NPT)])


def _edge_phase(xl, xr, src, dst, att):
    f = pl.kernel(
        _edge_body,
        out_type=[
            jax.ShapeDtypeStruct((E_PAD,), jnp.float32),
            jax.ShapeDtypeStruct((NC * N_P,), jnp.float32),
            jax.ShapeDtypeStruct((NC * N_P, HID), jnp.float32),
        ],
        mesh=plsc.VectorSubcoreMesh(core_axis_name="c", subcore_axis_name="s"),
        scratch_types=[
            pltpu.VMEM((EB,), jnp.int32),
            pltpu.VMEM((EB,), jnp.int32),
            pltpu.VMEM((EB, HID), jnp.float32),
            pltpu.VMEM((EB, HID), jnp.float32),
            pltpu.VMEM((HID,), jnp.float32),
            pltpu.VMEM((EB,), jnp.float32),
            pltpu.VMEM_SHARED((N_P,), jnp.float32),
            pltpu.VMEM_SHARED((N_P, HID), jnp.float32),
            pltpu.SemaphoreType.DMA,
            pltpu.SemaphoreType.DMA,
        ],
    )
    return f(xl, xr, src, dst, att)


def _alpha_body(ex0, ex1, ex2, dst_hbm, d0, d1, d2,
                al0, al1, al2,
                dst_f, ex_f, dv_f, al_f, sem):
    cid = lax.axis_index("c")
    sid = lax.axis_index("s")
    base = (cid * NS + sid) * EPW

    pltpu.sync_copy(dst_hbm.at[pl.ds(base, EPW)], dst_f)
    for ex_hbm, dsum_hbm, al_hbm in ((ex0, d0, al0), (ex1, d1, al1),
                                     (ex2, d2, al2)):
        pltpu.sync_copy(ex_hbm.at[pl.ds(base, EPW)], ex_f)

        def fire(i, carry):
            pltpu.async_copy(dsum_hbm.at[dst_f.at[pl.ds(i * EB, EB)]],
                             dv_f.at[pl.ds(i * EB, EB)], sem)
            return carry
        lax.fori_loop(0, NB, fire, 0)

        def drain(i, carry):
            pltpu.make_async_copy(dsum_hbm.at[dst_f.at[pl.ds(i * EB, EB)]],
                                  dv_f.at[pl.ds(i * EB, EB)], sem).wait()
            return carry
        lax.fori_loop(0, NB, drain, 0)

        def comp(i, carry):
            ex16 = ex_f[pl.ds(i * 16, 16)]
            d16 = dv_f[pl.ds(i * 16, 16)]
            al_f[pl.ds(i * 16, 16)] = ex16 / (d16 + 1e-16)
            return carry
        lax.fori_loop(0, EPW // 16, comp, 0)
        pltpu.sync_copy(al_f, al_hbm.at[pl.ds(base, EPW)])


def _alpha3(exs, dst, dsums):
    f = pl.kernel(
        _alpha_body,
        out_type=[jax.ShapeDtypeStruct((E_PAD,), jnp.float32)] * 3,
        mesh=plsc.VectorSubcoreMesh(core_axis_name="c", subcore_axis_name="s"),
        scratch_types=[
            pltpu.VMEM((EPW,), jnp.int32),
            pltpu.VMEM((EPW,), jnp.float32),
            pltpu.VMEM((EPW,), jnp.float32),
            pltpu.VMEM((EPW,), jnp.float32),
            pltpu.SemaphoreType.DMA,
        ],
    )
    return f(exs[0], exs[1], exs[2], dst, dsums[0], dsums[1], dsums[2])


def _prologue_body(x_ref, Win_ref, bin_ref, Wl_ref, bl_ref, Wr_ref, br_ref,
                   xl_ref, xr_ref):
    h = jnp.maximum(
        jnp.dot(x_ref[...], Win_ref[...].T, preferred_element_type=jnp.float32)
        + bin_ref[...], 0.0)
    xl_ref[...] = jnp.dot(h, Wl_ref[...].T, preferred_element_type=jnp.float32) + bl_ref[...]
    xr_ref[...] = jnp.dot(h, Wr_ref[...].T, preferred_element_type=jnp.float32) + br_ref[...]


def _prologue(x, W_in, b_in, Wl, bl, Wr, br):
    grid = (N_P // ROW_BLK,)
    blk = lambda i: (i, 0)
    full = lambda i: (0, 0)
    return pl.pallas_call(
        _prologue_body,
        grid=grid,
        in_specs=[
            pl.BlockSpec((ROW_BLK, HID), blk),
            pl.BlockSpec((HID, HID), full),
            pl.BlockSpec((1, HID), full),
            pl.BlockSpec((HID, HID), full),
            pl.BlockSpec((1, HID), full),
            pl.BlockSpec((HID, HID), full),
            pl.BlockSpec((1, HID), full),
        ],
        out_specs=[
            pl.BlockSpec((ROW_BLK, HID), blk),
            pl.BlockSpec((ROW_BLK, HID), blk),
        ],
        out_shape=[
            jax.ShapeDtypeStruct((N_P, HID), jnp.float32),
            jax.ShapeDtypeStruct((N_P, HID), jnp.float32),
        ],
    )(x, W_in, b_in.reshape(1, HID), Wl, bl.reshape(1, HID), Wr, br.reshape(1, HID))


def _combine(o0_ref, o1_ref, den_ref):
    s = jnp.sum(den_ref[...], axis=(0, 1))
    inv = 1.0 / (s + 1e-16)
    return (o0_ref[...] + o1_ref[...]) * inv[:, None], s


def _mid_body(o0_ref, o1_ref, den_ref, gb_ref, Wl_ref, bl_ref, Wr_ref, br_ref,
              xl_ref, xr_ref, dsum_ref):
    out, s = _combine(o0_ref, o1_ref, den_ref)
    dsum_ref[...] = s[None, None, :]
    v = out + gb_ref[...]
    h = jnp.where(v > 0.0, v, jnp.exp(jnp.minimum(v, 0.0)) - 1.0)
    xl_ref[...] = jnp.dot(h, Wl_ref[...].T, preferred_element_type=jnp.float32) + bl_ref[...]
    xr_ref[...] = jnp.dot(h, Wr_ref[...].T, preferred_element_type=jnp.float32) + br_ref[...]


def _mid(outu, den, gb, Wl, bl, Wr, br):
    grid = (N_P // ROW_BLK,)
    blk = lambda i: (i, 0)
    blk1 = lambda i: (i + N_P // ROW_BLK, 0)
    dblk = lambda i: (0, 0, i)
    full = lambda i: (0, 0)
    return pl.pallas_call(
        _mid_body,
        grid=grid,
        in_specs=[
            pl.BlockSpec((ROW_BLK, HID), blk),
            pl.BlockSpec((ROW_BLK, HID), blk1),
            pl.BlockSpec((NC, 1, ROW_BLK), dblk),
            pl.BlockSpec((1, HID), full),
            pl.BlockSpec((HID, HID), full),
            pl.BlockSpec((1, HID), full),
            pl.BlockSpec((HID, HID), full),
            pl.BlockSpec((1, HID), full),
        ],
        out_specs=[
            pl.BlockSpec((ROW_BLK, HID), blk),
            pl.BlockSpec((ROW_BLK, HID), blk),
            pl.BlockSpec((1, 1, ROW_BLK), dblk),
        ],
        out_shape=[
            jax.ShapeDtypeStruct((N_P, HID), jnp.float32),
            jax.ShapeDtypeStruct((N_P, HID), jnp.float32),
            jax.ShapeDtypeStruct((1, 1, N_P), jnp.float32),
        ],
    )(outu, outu, den, gb.reshape(1, HID), Wl, bl.reshape(1, HID),
      Wr, br.reshape(1, HID))


def _epilogue_body(o0_ref, o1_ref, den_ref, gb_ref, Wih_ref, bih_ref,
                   Wout_ref, bout_ref,
                   emb_ref, out_ref, dsum_ref, acc_ref):
    i = pl.program_id(0)
    out, s = _combine(o0_ref, o1_ref, den_ref)
    dsum_ref[...] = s[None, None, :]
    h = out + gb_ref[...]
    emb_ref[...] = h
    gates = jnp.dot(h, Wih_ref[...].T, preferred_element_type=jnp.float32) + bih_ref[...]
    i_g = gates[:, 0 * HID:1 * HID]
    g_g = gates[:, 2 * HID:3 * HID]
    o_g = gates[:, 3 * HID:4 * HID]
    c = jax.nn.sigmoid(i_g) * jnp.tanh(g_g)
    ht = jax.nn.sigmoid(o_g) * jnp.tanh(c)
    rows = lax.broadcasted_iota(jnp.int32, (ROW_BLK, 1), 0) + i * ROW_BLK
    ht = jnp.where(rows < N_NODES, ht, 0.0)

    @pl.when(i == 0)
    def _():
        acc_ref[...] = jnp.zeros_like(acc_ref)

    acc_ref[...] += jnp.sum(ht, axis=0, keepdims=True)

    @pl.when(i == pl.num_programs(0) - 1)
    def _():
        xg = acc_ref[...] * (1.0 / N_NODES)
        out_ref[...] = jnp.dot(xg, Wout_ref[...].T,
                               preferred_element_type=jnp.float32) + bout_ref[...]


def _epilogue(outu, den, gb, W_ih, b_ih, b_hh, W_out, b_out):
    grid = (N_P // ROW_BLK,)
    blk = lambda i: (i, 0)
    blk1 = lambda i: (i + N_P // ROW_BLK, 0)
    dblk = lambda i: (0, 0, i)
    full = lambda i: (0, 0)
    return pl.pallas_call(
        _epilogue_body,
        grid=grid,
        in_specs=[
            pl.BlockSpec((ROW_BLK, HID), blk),
            pl.BlockSpec((ROW_BLK, HID), blk1),
            pl.BlockSpec((NC, 1, ROW_BLK), dblk),
            pl.BlockSpec((1, HID), full),
            pl.BlockSpec((4 * HID, HID), full),
            pl.BlockSpec((1, 4 * HID), full),
            pl.BlockSpec((OUT_CH, HID), full),
            pl.BlockSpec((1, OUT_CH), full),
        ],
        out_specs=[
            pl.BlockSpec((ROW_BLK, HID), blk),
            pl.BlockSpec((1, OUT_CH), full),
            pl.BlockSpec((1, 1, ROW_BLK), dblk),
        ],
        out_shape=[
            jax.ShapeDtypeStruct((N_P, HID), jnp.float32),
            jax.ShapeDtypeStruct((1, OUT_CH), jnp.float32),
            jax.ShapeDtypeStruct((1, 1, N_P), jnp.float32),
        ],
        scratch_shapes=[pltpu.VMEM((1, HID), jnp.float32)],
    )(outu, outu, den, gb.reshape(1, HID), W_ih,
      (b_ih + b_hh).reshape(1, 4 * HID), W_out, b_out.reshape(1, OUT_CH))


def kernel(x, edge_index, W_in, b_in, lin_l_W, lin_l_b, lin_r_W, lin_r_b,
           att_W, gat_b, W_ih, W_hh, b_ih, b_hh, W_out, b_out):
    del W_hh
    N = x.shape[0]
    loop = jnp.arange(N, dtype=jnp.int32)
    pad = jnp.zeros((E_PAD - E_VALID,), jnp.int32)
    src = jnp.concatenate([edge_index[0].astype(jnp.int32), loop, pad])
    dst = jnp.concatenate([edge_index[1].astype(jnp.int32), loop, pad])
    x_p = jnp.concatenate(
        [x, jnp.zeros((N_P - N_NODES, HID), jnp.float32)], axis=0)

    xl, xr = _prologue(x_p, W_in, b_in, lin_l_W[0], lin_l_b[0], lin_r_W[0],
                       lin_r_b[0])
    exs, dsums = [], []
    for i in range(N_LAYERS):
        ex, den, outu = _edge_phase(xl, xr, src, dst, att_W[i, 0])
        den = den.reshape(NC, 1, N_P)
        exs.append(ex)
        if i < N_LAYERS - 1:
            xl, xr, dsum = _mid(outu, den, gat_b[i], lin_l_W[i + 1],
                                lin_l_b[i + 1], lin_r_W[i + 1],
                                lin_r_b[i + 1])
        else:
            emb, final, dsum = _epilogue(outu, den, gat_b[i], W_ih, b_ih,
                                         b_hh, W_out, b_out)
        dsums.append(dsum.reshape(N_P))
    als = _alpha3(exs, dst, dsums)
    attn = tuple(a[:E_VALID, None] for a in als)
    return (final, emb[:N_NODES]) + attn
